# baseline (speedup 1.0000x reference)
import functools

import jax
import jax.numpy as jnp
from jax import lax
from jax.experimental import pallas as pl
from jax.experimental.pallas import tpu as pltpu
from jax.experimental.pallas import tpu_sc as plsc

B = 4096
M = 50
MP = 64
D = 64

NC = 2
NS = 16
NW = NC * NS

ROWS = B * MP
RPW = ROWS // NW
CH = 128
NCHUNK = RPW // CH
BPW = B // NW
NBUF = 4


def _sc_gather(mids2d, item_ids, gro_ids, u2e_w, v2e_w, g2e_w):
    mesh = plsc.VectorSubcoreMesh(core_axis_name="c", subcore_axis_name="s")

    @functools.partial(
        pl.kernel,
        mesh=mesh,
        out_type=[
            jax.ShapeDtypeStruct((ROWS, D), jnp.float32),
            jax.ShapeDtypeStruct((B, D), jnp.float32),
            jax.ShapeDtypeStruct((B, D), jnp.float32),
        ],
        scratch_types=[
            pltpu.VMEM((NCHUNK, CH), jnp.int32),
            [pltpu.VMEM((CH, D), jnp.float32) for _ in range(NBUF)],
            pltpu.VMEM((BPW,), jnp.int32),
            pltpu.VMEM((BPW, D), jnp.float32),
            [pltpu.SemaphoreType.DMA for _ in range(NBUF)],
            [pltpu.SemaphoreType.DMA for _ in range(NBUF)],
        ],
        compiler_params=pltpu.CompilerParams(use_tc_tiling_on_sc=False),
    )
    def k(mids_hbm, iids_hbm, gids_hbm, u2e_hbm, v2e_hbm, g2e_hbm,
          memb_out, item_out, group_out,
          idx_v, bufs, sid_v, rows_v, gsems, osems):
        wid = lax.axis_index("s") * NC + lax.axis_index("c")
        base = wid * RPW

        pltpu.sync_copy(mids_hbm.at[pl.ds(wid * NCHUNK, NCHUNK)], idx_v)

        def gather_start(c, b):
            pltpu.async_copy(u2e_hbm.at[idx_v.at[c]], bufs[b], gsems[b])

        def copyout_start(c, b):
            pltpu.async_copy(bufs[b],
                             memb_out.at[pl.ds(base + c * CH, CH)], osems[b])

        for b in range(NBUF):
            gather_start(b, b)

        def body(j):
            for b in range(NBUF):
                pltpu.make_async_copy(
                    u2e_hbm.at[idx_v.at[0]], bufs[b], gsems[b]).wait()
                copyout_start(j + b, b)
            for b in range(NBUF):
                nxt = j + b + NBUF

                @pl.when(nxt < NCHUNK)
                def _():
                    pltpu.make_async_copy(
                        bufs[b], memb_out.at[pl.ds(base, CH)], osems[b]).wait()
                    gather_start(nxt, b)

        pl.loop(0, NCHUNK, step=NBUF)(body)

        for b in range(NBUF):
            pltpu.make_async_copy(
                bufs[b], memb_out.at[pl.ds(base, CH)], osems[b]).wait()

        sbase = wid * BPW
        pltpu.sync_copy(iids_hbm.at[pl.ds(sbase, BPW)], sid_v)
        pltpu.async_copy(v2e_hbm.at[sid_v], rows_v, gsems[0]).wait()
        pltpu.sync_copy(rows_v, item_out.at[pl.ds(sbase, BPW)])

        pltpu.sync_copy(gids_hbm.at[pl.ds(sbase, BPW)], sid_v)
        pltpu.async_copy(g2e_hbm.at[sid_v], rows_v, gsems[0]).wait()
        pltpu.sync_copy(rows_v, group_out.at[pl.ds(sbase, BPW)])

    return k(mids2d, item_ids, gro_ids, u2e_w, v2e_w, g2e_w)


BG = 8
GRID = B // BG


def _tc_attn_body(emb_ref, maskf_ref, mask_ref, item_ref, group_ref,
                  wq_ref, bq_ref, wk_ref, bk_ref, wv_ref, bv_ref,
                  w1a_ref, w1b_ref, b1_ref, w2_ref, b2_ref, out_ref):
    maskf = maskf_ref[:]
    emb = jnp.where(maskf > 0.0, emb_ref[:], 0.0)
    q = (jnp.dot(emb, wq_ref[:], preferred_element_type=jnp.float32)
         + bq_ref[:]) * maskf
    k = (jnp.dot(emb, wk_ref[:], preferred_element_type=jnp.float32)
         + bk_ref[:]) * maskf
    v = jnp.dot(emb, wv_ref[:], preferred_element_type=jnp.float32) + bv_ref[:]

    rows = []
    for g in range(BG):
        s0, s1 = g * MP, (g + 1) * MP
        qg, kg, vg = q[s0:s1], k[s0:s1], v[s0:s1]
        eg = emb[s0:s1]
        mrow = maskf[s0:s1]
        mcol = mask_ref[g:g + 1, :]
        energy = lax.dot_general(qg, kg, (((1,), (1,)), ((), ())),
                                 preferred_element_type=jnp.float32)
        energy = jnp.clip(energy, -50.0, 50.0)
        eexp = jnp.exp(energy) * mcol
        attn = eexp / jnp.sum(eexp, axis=1, keepdims=True)
        mo = jnp.dot(attn, vg, preferred_element_type=jnp.float32)
        overall = 0.5 * (mo * mrow) + 0.5 * eg
        ipart = jnp.dot(item_ref[g:g + 1, :], w1b_ref[:],
                        preferred_element_type=jnp.float32)
        h = jnp.maximum(
            jnp.dot(overall, w1a_ref[:], preferred_element_type=jnp.float32)
            + mrow * ipart + b1_ref[:], 0.0)
        a = jnp.dot(h, w2_ref[:], preferred_element_type=jnp.float32) + b2_ref[:]
        a = jnp.clip(a, -50.0, 50.0)
        aexp = jnp.exp(a) * mrow
        w = aexp / jnp.sum(aexp)
        pooled = jnp.sum(w * overall, axis=0, keepdims=True)
        rows.append(0.5 * pooled + 0.5 * group_ref[g:g + 1, :])
    out_ref[:] = jnp.concatenate(rows, axis=0)


def _tc_attn(emb_flat, maskf, mask2d, item_emb, group_emb,
             Wq, bq, Wk, bk, Wv, bv, W1a, W1b, b1, W2, b2):
    full = lambda shape: pl.BlockSpec(shape, lambda i: (0, 0))
    return pl.pallas_call(
        _tc_attn_body,
        grid=(GRID,),
        in_specs=[
            pl.BlockSpec((BG * MP, D), lambda i: (i, 0)),
            pl.BlockSpec((BG * MP, 1), lambda i: (i, 0)),
            pl.BlockSpec((BG, MP), lambda i: (i, 0)),
            pl.BlockSpec((BG, D), lambda i: (i, 0)),
            pl.BlockSpec((BG, D), lambda i: (i, 0)),
            full((D, D)), full((1, D)),
            full((D, D)), full((1, D)),
            full((D, D)), full((1, D)),
            full((D, 16)), full((D, 16)), full((1, 16)),
            full((16, 1)), full((1, 1)),
        ],
        out_specs=pl.BlockSpec((BG, D), lambda i: (i, 0)),
        out_shape=jax.ShapeDtypeStruct((B, D), jnp.float32),
    )(emb_flat, maskf, mask2d, item_emb, group_emb,
      Wq, bq, Wk, bk, Wv, bv, W1a, W1b, b1, W2, b2)


def kernel(gro_inputs, item_inputs, menb_ids, mask, u2e_w, v2e_w, g2e_w,
           Wq, bq, Wk, bk, Wv, bv, W1, b1, W2, b2):
    mids_p = jnp.pad(menb_ids, ((0, 0), (0, MP - M)))
    mids2d = mids_p.reshape(ROWS // CH, CH)
    memb_flat, item_emb, group_emb = _sc_gather(
        mids2d, item_inputs, gro_inputs, u2e_w, v2e_w, g2e_w)

    mask_p = jnp.pad(mask, ((0, 0), (0, MP - M)))
    maskf = mask_p.reshape(B * MP, 1)
    return _tc_attn(memb_flat, maskf, mask_p, item_emb, group_emb,
                    Wq, bq.reshape(1, D), Wk, bk.reshape(1, D),
                    Wv, bv.reshape(1, D),
                    W1[:D], W1[D:], b1.reshape(1, 16),
                    W2, b2.reshape(1, 1))


# speedup vs baseline: 2.2021x; 2.2021x over previous
import functools

import jax
import jax.numpy as jnp
from jax import lax
from jax.experimental import pallas as pl
from jax.experimental.pallas import tpu as pltpu
from jax.experimental.pallas import tpu_sc as plsc

B = 4096
M = 50
MP = 64
D = 64
NU = 100000

NC = 2
NS = 16
NW = NC * NS

ROWS = B * MP
RPW = ROWS // NW
CH = 128
NCHUNK = RPW // CH
BPW = B // NW
NBUF = 4


def _sc_gather(mids2d, item_ids, gro_ids, u2e_w, v2e_w, g2e_w):
    mesh = plsc.VectorSubcoreMesh(core_axis_name="c", subcore_axis_name="s")

    @functools.partial(
        pl.kernel,
        mesh=mesh,
        out_type=[
            jax.ShapeDtypeStruct((ROWS, D), jnp.float32),
            jax.ShapeDtypeStruct((B, D), jnp.float32),
            jax.ShapeDtypeStruct((B, D), jnp.float32),
        ],
        scratch_types=[
            pltpu.VMEM((NCHUNK, CH), jnp.int32),
            [pltpu.VMEM((CH, D), jnp.float32) for _ in range(NBUF)],
            pltpu.VMEM((BPW,), jnp.int32),
            pltpu.VMEM((BPW, D), jnp.float32),
            [pltpu.SemaphoreType.DMA for _ in range(NBUF)],
            [pltpu.SemaphoreType.DMA for _ in range(NBUF)],
        ],
        compiler_params=pltpu.CompilerParams(use_tc_tiling_on_sc=False),
    )
    def k(mids_hbm, iids_hbm, gids_hbm, u2e_hbm, v2e_hbm, g2e_hbm,
          memb_out, item_out, group_out,
          idx_v, bufs, sid_v, rows_v, gsems, osems):
        wid = lax.axis_index("s") * NC + lax.axis_index("c")
        base = wid * RPW

        pltpu.sync_copy(mids_hbm.at[pl.ds(wid * NCHUNK, NCHUNK)], idx_v)

        def gather_start(c, b):
            pltpu.async_copy(u2e_hbm.at[idx_v.at[c]], bufs[b], gsems[b])

        def copyout_start(c, b):
            pltpu.async_copy(bufs[b],
                             memb_out.at[pl.ds(base + c * CH, CH)], osems[b])

        for b in range(NBUF):
            gather_start(b, b)

        def body(j):
            for b in range(NBUF):
                pltpu.make_async_copy(
                    u2e_hbm.at[idx_v.at[0]], bufs[b], gsems[b]).wait()
                copyout_start(j + b, b)
            for b in range(NBUF):
                nxt = j + b + NBUF

                @pl.when(nxt < NCHUNK)
                def _():
                    pltpu.make_async_copy(
                        bufs[b], memb_out.at[pl.ds(base, CH)], osems[b]).wait()
                    gather_start(nxt, b)

        pl.loop(0, NCHUNK, step=NBUF)(body)

        for b in range(NBUF):
            pltpu.make_async_copy(
                bufs[b], memb_out.at[pl.ds(base, CH)], osems[b]).wait()

        sbase = wid * BPW
        pltpu.sync_copy(iids_hbm.at[pl.ds(sbase, BPW)], sid_v)
        pltpu.async_copy(v2e_hbm.at[sid_v], rows_v, gsems[0]).wait()
        pltpu.sync_copy(rows_v, item_out.at[pl.ds(sbase, BPW)])

        pltpu.sync_copy(gids_hbm.at[pl.ds(sbase, BPW)], sid_v)
        pltpu.async_copy(g2e_hbm.at[sid_v], rows_v, gsems[0]).wait()
        pltpu.sync_copy(rows_v, group_out.at[pl.ds(sbase, BPW)])

    return k(mids2d, item_ids, gro_ids, u2e_w, v2e_w, g2e_w)


BG = 8
GRID = B // BG


def _tc_attn_body(emb_ref, maskf_ref, mask_ref, item_ref, group_ref,
                  wq_ref, bq_ref, wk_ref, bk_ref, wv_ref, bv_ref,
                  w1a_ref, w1b_ref, b1_ref, w2_ref, b2_ref, out_ref):
    maskf = maskf_ref[:]
    emb = jnp.where(maskf > 0.0, emb_ref[:], 0.0)
    q = (jnp.dot(emb, wq_ref[:], preferred_element_type=jnp.float32)
         + bq_ref[:]) * maskf
    k = (jnp.dot(emb, wk_ref[:], preferred_element_type=jnp.float32)
         + bk_ref[:]) * maskf
    v = jnp.dot(emb, wv_ref[:], preferred_element_type=jnp.float32) + bv_ref[:]

    rows = []
    for g in range(BG):
        s0, s1 = g * MP, (g + 1) * MP
        qg, kg, vg = q[s0:s1], k[s0:s1], v[s0:s1]
        eg = emb[s0:s1]
        mrow = maskf[s0:s1]
        mcol = mask_ref[g:g + 1, :]
        energy = lax.dot_general(qg, kg, (((1,), (1,)), ((), ())),
                                 preferred_element_type=jnp.float32)
        energy = jnp.clip(energy, -50.0, 50.0)
        eexp = jnp.exp(energy) * mcol
        attn = eexp / jnp.sum(eexp, axis=1, keepdims=True)
        mo = jnp.dot(attn, vg, preferred_element_type=jnp.float32)
        overall = 0.5 * (mo * mrow) + 0.5 * eg
        ipart = jnp.dot(item_ref[g:g + 1, :], w1b_ref[:],
                        preferred_element_type=jnp.float32)
        h = jnp.maximum(
            jnp.dot(overall, w1a_ref[:], preferred_element_type=jnp.float32)
            + mrow * ipart + b1_ref[:], 0.0)
        a = jnp.dot(h, w2_ref[:], preferred_element_type=jnp.float32) + b2_ref[:]
        a = jnp.clip(a, -50.0, 50.0)
        aexp = jnp.exp(a) * mrow
        w = aexp / jnp.sum(aexp)
        pooled = jnp.sum(w * overall, axis=0, keepdims=True)
        rows.append(0.5 * pooled + 0.5 * group_ref[g:g + 1, :])
    out_ref[:] = jnp.concatenate(rows, axis=0)


def _tc_attn(emb_flat, maskf, mask2d, item_emb, group_emb,
             Wq, bq, Wk, bk, Wv, bv, W1a, W1b, b1, W2, b2):
    full = lambda shape: pl.BlockSpec(shape, lambda i: (0, 0))
    return pl.pallas_call(
        _tc_attn_body,
        grid=(GRID,),
        in_specs=[
            pl.BlockSpec((BG * MP, D), lambda i: (i, 0)),
            pl.BlockSpec((BG * MP, 1), lambda i: (i, 0)),
            pl.BlockSpec((BG, MP), lambda i: (i, 0)),
            pl.BlockSpec((BG, D), lambda i: (i, 0)),
            pl.BlockSpec((BG, D), lambda i: (i, 0)),
            full((D, D)), full((1, D)),
            full((D, D)), full((1, D)),
            full((D, D)), full((1, D)),
            full((D, 16)), full((D, 16)), full((1, 16)),
            full((16, 1)), full((1, 1)),
        ],
        out_specs=pl.BlockSpec((BG, D), lambda i: (i, 0)),
        out_shape=jax.ShapeDtypeStruct((B, D), jnp.float32),
    )(emb_flat, maskf, mask2d, item_emb, group_emb,
      Wq, bq, Wk, bk, Wv, bv, W1a, W1b, b1, W2, b2)


def kernel(gro_inputs, item_inputs, menb_ids, mask, u2e_w, v2e_w, g2e_w,
           Wq, bq, Wk, bk, Wv, bv, W1, b1, W2, b2):
    mask_pb = jnp.pad(mask, ((0, 0), (0, MP - M))) > 0.0
    mids_p = jnp.pad(menb_ids, ((0, 0), (0, MP - M)))
    spread = jax.lax.broadcasted_iota(jnp.int32, (B, MP), 0) * MP \
        + jax.lax.broadcasted_iota(jnp.int32, (B, MP), 1)
    mids_p = jnp.where(mask_pb, mids_p, spread % NU)
    mids2d = mids_p.reshape(ROWS // CH, CH)
    memb_flat, item_emb, group_emb = _sc_gather(
        mids2d, item_inputs, gro_inputs, u2e_w, v2e_w, g2e_w)

    mask_p = jnp.pad(mask, ((0, 0), (0, MP - M)))
    maskf = mask_p.reshape(B * MP, 1)
    return _tc_attn(memb_flat, maskf, mask_p, item_emb, group_emb,
                    Wq, bq.reshape(1, D), Wk, bk.reshape(1, D),
                    Wv, bv.reshape(1, D),
                    W1[:D], W1[D:], b1.reshape(1, 16),
                    W2, b2.reshape(1, 1))


# speedup vs baseline: 4.7039x; 2.1361x over previous
import functools

import jax
import jax.numpy as jnp
from jax import lax
from jax.experimental import pallas as pl
from jax.experimental.pallas import tpu as pltpu
from jax.experimental.pallas import tpu_sc as plsc

B = 4096
M = 50
MP = 64
D = 64
NU = 100000

NC = 2
NS = 16
NW = NC * NS

ROWS = B * MP
RPW = ROWS // NW
CH = 128
NCHUNK = RPW // CH
BPW = B // NW
NBUF = 4


def _sc_gather(mids2d, item_ids, gro_ids, u2e_w, v2e_w, g2e_w):
    mesh = plsc.VectorSubcoreMesh(core_axis_name="c", subcore_axis_name="s")

    @functools.partial(
        pl.kernel,
        mesh=mesh,
        out_type=[
            jax.ShapeDtypeStruct((ROWS, D), jnp.float32),
            jax.ShapeDtypeStruct((B, D), jnp.float32),
            jax.ShapeDtypeStruct((B, D), jnp.float32),
        ],
        scratch_types=[
            pltpu.VMEM((NCHUNK, CH), jnp.int32),
            [pltpu.VMEM((CH, D), jnp.float32) for _ in range(NBUF)],
            pltpu.VMEM((BPW,), jnp.int32),
            pltpu.VMEM((BPW, D), jnp.float32),
            [pltpu.SemaphoreType.DMA for _ in range(NBUF)],
            [pltpu.SemaphoreType.DMA for _ in range(NBUF)],
        ],
        compiler_params=pltpu.CompilerParams(use_tc_tiling_on_sc=False),
    )
    def k(mids_hbm, iids_hbm, gids_hbm, u2e_hbm, v2e_hbm, g2e_hbm,
          memb_out, item_out, group_out,
          idx_v, bufs, sid_v, rows_v, gsems, osems):
        wid = lax.axis_index("s") * NC + lax.axis_index("c")
        base = wid * RPW

        pltpu.sync_copy(mids_hbm.at[pl.ds(wid * NCHUNK, NCHUNK)], idx_v)

        def gather_start(c, b):
            pltpu.async_copy(u2e_hbm.at[idx_v.at[c]], bufs[b], gsems[b])

        def copyout_start(c, b):
            pltpu.async_copy(bufs[b],
                             memb_out.at[pl.ds(base + c * CH, CH)], osems[b])

        for b in range(NBUF):
            gather_start(b, b)

        def body(j):
            for b in range(NBUF):
                pltpu.make_async_copy(
                    u2e_hbm.at[idx_v.at[0]], bufs[b], gsems[b]).wait()
                copyout_start(j + b, b)
            for b in range(NBUF):
                nxt = j + b + NBUF

                @pl.when(nxt < NCHUNK)
                def _():
                    pltpu.make_async_copy(
                        bufs[b], memb_out.at[pl.ds(base, CH)], osems[b]).wait()
                    gather_start(nxt, b)

        pl.loop(0, NCHUNK, step=NBUF)(body)

        for b in range(NBUF):
            pltpu.make_async_copy(
                bufs[b], memb_out.at[pl.ds(base, CH)], osems[b]).wait()

        sbase = wid * BPW
        pltpu.sync_copy(iids_hbm.at[pl.ds(sbase, BPW)], sid_v)
        pltpu.async_copy(v2e_hbm.at[sid_v], rows_v, gsems[0]).wait()
        pltpu.sync_copy(rows_v, item_out.at[pl.ds(sbase, BPW)])

        pltpu.sync_copy(gids_hbm.at[pl.ds(sbase, BPW)], sid_v)
        pltpu.async_copy(g2e_hbm.at[sid_v], rows_v, gsems[0]).wait()
        pltpu.sync_copy(rows_v, group_out.at[pl.ds(sbase, BPW)])

    return k(mids2d, item_ids, gro_ids, u2e_w, v2e_w, g2e_w)


BG = 32
GRID = B // BG


def _tc_attn_body(emb_ref, maskf_ref, mask_ref, item_ref, group_ref,
                  wq_ref, bq_ref, wk_ref, bk_ref, wve_ref, bve_ref,
                  w1a_ref, w1b_ref, b1_ref, w2_ref, b2_ref,
                  sel_ref, selt_ref, out_ref):
    maskf = maskf_ref[:]
    emb = jnp.where(maskf > 0.0, emb_ref[:], 0.0)
    q = (jnp.dot(emb, wq_ref[:], preferred_element_type=jnp.float32)
         + bq_ref[:]) * maskf
    k = (jnp.dot(emb, wk_ref[:], preferred_element_type=jnp.float32)
         + bk_ref[:]) * maskf
    v = (jnp.dot(emb, wve_ref[:], preferred_element_type=jnp.float32)
         + bve_ref[:])

    parts = []
    for g in range(BG):
        s0, s1 = g * MP, (g + 1) * MP
        qg, kg, vg = q[s0:s1], k[s0:s1], v[s0:s1]
        mrow = maskf[s0:s1]
        mcol = mask_ref[g:g + 1, :]
        energy = lax.dot_general(qg, kg, (((1,), (1,)), ((), ())),
                                 preferred_element_type=jnp.float32)
        energy = jnp.clip(energy, -50.0, 50.0)
        eexp = jnp.exp(energy) * mcol
        we = jnp.dot(eexp, vg, preferred_element_type=jnp.float32)
        mo = we[:, :D] / we[:, D:D + 1]
        parts.append(0.5 * (mo * mrow) + 0.5 * emb[s0:s1])
    overall = jnp.concatenate(parts, axis=0)

    ipart8 = jnp.dot(item_ref[:], w1b_ref[:],
                     preferred_element_type=jnp.float32)
    ipart = jnp.dot(selt_ref[:], ipart8,
                    preferred_element_type=jnp.float32)
    h = jnp.maximum(
        jnp.dot(overall, w1a_ref[:], preferred_element_type=jnp.float32)
        + maskf * ipart + b1_ref[:], 0.0)
    a = jnp.dot(h, w2_ref[:], preferred_element_type=jnp.float32) + b2_ref[:]
    a = jnp.clip(a, -50.0, 50.0)
    aexp = jnp.exp(a) * maskf

    num = jnp.dot(sel_ref[:], aexp * overall,
                  preferred_element_type=jnp.float32)
    den = jnp.dot(sel_ref[:], aexp,
                  preferred_element_type=jnp.float32)
    out_ref[:] = 0.5 * (num / den) + 0.5 * group_ref[:]


def _tc_attn(emb_flat, maskf, mask2d, item_emb, group_emb,
             Wq, bq, Wk, bk, Wve, bve, W1a, W1b, b1, W2, b2, sel, selt):
    full = lambda s: pl.BlockSpec(s, lambda i: (0, 0))
    return pl.pallas_call(
        _tc_attn_body,
        grid=(GRID,),
        in_specs=[
            pl.BlockSpec((BG * MP, D), lambda i: (i, 0)),
            pl.BlockSpec((BG * MP, 1), lambda i: (i, 0)),
            pl.BlockSpec((BG, MP), lambda i: (i, 0)),
            pl.BlockSpec((BG, D), lambda i: (i, 0)),
            pl.BlockSpec((BG, D), lambda i: (i, 0)),
            full((D, D)), full((1, D)),
            full((D, D)), full((1, D)),
            full((D, D + 1)), full((1, D + 1)),
            full((D, 16)), full((D, 16)), full((1, 16)),
            full((16, 1)), full((1, 1)),
            full((BG, BG * MP)), full((BG * MP, BG)),
        ],
        out_specs=pl.BlockSpec((BG, D), lambda i: (i, 0)),
        out_shape=jax.ShapeDtypeStruct((B, D), jnp.float32),
    )(emb_flat, maskf, mask2d, item_emb, group_emb,
      Wq, bq, Wk, bk, Wve, bve, W1a, W1b, b1, W2, b2, sel, selt)


def kernel(gro_inputs, item_inputs, menb_ids, mask, u2e_w, v2e_w, g2e_w,
           Wq, bq, Wk, bk, Wv, bv, W1, b1, W2, b2):
    mask_pb = jnp.pad(mask, ((0, 0), (0, MP - M))) > 0.0
    mids_p = jnp.pad(menb_ids, ((0, 0), (0, MP - M)))
    spread = jax.lax.broadcasted_iota(jnp.int32, (B, MP), 0) * MP \
        + jax.lax.broadcasted_iota(jnp.int32, (B, MP), 1)
    mids_p = jnp.where(mask_pb, mids_p, spread % NU)
    mids2d = mids_p.reshape(ROWS // CH, CH)
    memb_flat, item_emb, group_emb = _sc_gather(
        mids2d, item_inputs, gro_inputs, u2e_w, v2e_w, g2e_w)

    mask_p = jnp.pad(mask, ((0, 0), (0, MP - M)))
    maskf = mask_p.reshape(B * MP, 1)
    Wve = jnp.pad(Wv, ((0, 0), (0, 1)))
    bve = jnp.pad(bv, ((0, 1)), constant_values=1.0).reshape(1, D + 1)
    gid = jax.lax.broadcasted_iota(jnp.int32, (BG, BG * MP), 1) // MP
    sel = (gid == jax.lax.broadcasted_iota(jnp.int32, (BG, BG * MP), 0)
           ).astype(jnp.float32)
    return _tc_attn(memb_flat, maskf, mask_p, item_emb, group_emb,
                    Wq, bq.reshape(1, D), Wk, bk.reshape(1, D),
                    Wve, bve,
                    W1[:D], W1[D:], b1.reshape(1, 16),
                    W2, b2.reshape(1, 1), sel, sel.T)


# speedup vs baseline: 8.1364x; 1.7297x over previous
import functools

import jax
import jax.numpy as jnp
from jax import lax
from jax.experimental import pallas as pl
from jax.experimental.pallas import tpu as pltpu
from jax.experimental.pallas import tpu_sc as plsc

B = 4096
M = 50
MP = 64
D = 64
NU = 100000

NC = 2
NS = 16
NW = NC * NS

ROWS = B * MP
RPW = ROWS // NW
CH = 128
NCHUNK = RPW // CH
BPW = B // NW
NBUF = 4


def _sc_gather(mids2d, item_ids, gro_ids, u2e_w, v2e_w, g2e_w):
    mesh = plsc.VectorSubcoreMesh(core_axis_name="c", subcore_axis_name="s")

    @functools.partial(
        pl.kernel,
        mesh=mesh,
        out_type=[
            jax.ShapeDtypeStruct((ROWS, D), jnp.float32),
            jax.ShapeDtypeStruct((B, D), jnp.float32),
            jax.ShapeDtypeStruct((B, D), jnp.float32),
        ],
        scratch_types=[
            pltpu.VMEM((NCHUNK, CH), jnp.int32),
            [pltpu.VMEM((CH, D), jnp.float32) for _ in range(NBUF)],
            pltpu.VMEM((BPW,), jnp.int32),
            pltpu.VMEM((BPW, D), jnp.float32),
            [pltpu.SemaphoreType.DMA for _ in range(NBUF)],
            [pltpu.SemaphoreType.DMA for _ in range(NBUF)],
        ],
        compiler_params=pltpu.CompilerParams(use_tc_tiling_on_sc=False),
    )
    def k(mids_hbm, iids_hbm, gids_hbm, u2e_hbm, v2e_hbm, g2e_hbm,
          memb_out, item_out, group_out,
          idx_v, bufs, sid_v, rows_v, gsems, osems):
        wid = lax.axis_index("s") * NC + lax.axis_index("c")
        base = wid * RPW

        pltpu.sync_copy(mids_hbm.at[pl.ds(wid * NCHUNK, NCHUNK)], idx_v)

        def gather_start(c, b):
            pltpu.async_copy(u2e_hbm.at[idx_v.at[c]], bufs[b], gsems[b])

        def copyout_start(c, b):
            pltpu.async_copy(bufs[b],
                             memb_out.at[pl.ds(base + c * CH, CH)], osems[b])

        for b in range(NBUF):
            gather_start(b, b)

        def body(j):
            for b in range(NBUF):
                pltpu.make_async_copy(
                    u2e_hbm.at[idx_v.at[0]], bufs[b], gsems[b]).wait()
                copyout_start(j + b, b)
            for b in range(NBUF):
                nxt = j + b + NBUF

                @pl.when(nxt < NCHUNK)
                def _():
                    pltpu.make_async_copy(
                        bufs[b], memb_out.at[pl.ds(base, CH)], osems[b]).wait()
                    gather_start(nxt, b)

        pl.loop(0, NCHUNK, step=NBUF)(body)

        for b in range(NBUF):
            pltpu.make_async_copy(
                bufs[b], memb_out.at[pl.ds(base, CH)], osems[b]).wait()

        sbase = wid * BPW
        pltpu.sync_copy(iids_hbm.at[pl.ds(sbase, BPW)], sid_v)
        pltpu.async_copy(v2e_hbm.at[sid_v], rows_v, gsems[0]).wait()
        pltpu.sync_copy(rows_v, item_out.at[pl.ds(sbase, BPW)])

        pltpu.sync_copy(gids_hbm.at[pl.ds(sbase, BPW)], sid_v)
        pltpu.async_copy(g2e_hbm.at[sid_v], rows_v, gsems[0]).wait()
        pltpu.sync_copy(rows_v, group_out.at[pl.ds(sbase, BPW)])

    return k(mids2d, item_ids, gro_ids, u2e_w, v2e_w, g2e_w)


BG = 64
GRID = B // BG
PACK = 4
PM = PACK * MP


def _tc_attn_body(emb_ref, maskf_ref, item_ref, group_ref,
                  wq_ref, bq_ref, wk_ref, bk_ref, wve_ref, bve_ref,
                  w1a_ref, w1b_ref, b1_ref, w2_ref, b2_ref,
                  bd_ref, out_ref):
    maskf = maskf_ref[:]
    emb = jnp.where(maskf > 0.0, emb_ref[:], 0.0)
    q = jnp.dot(emb, wq_ref[:], preferred_element_type=jnp.float32) + bq_ref[:]
    k = jnp.dot(emb, wk_ref[:], preferred_element_type=jnp.float32) + bk_ref[:]
    v = (jnp.dot(emb, wve_ref[:], preferred_element_type=jnp.float32)
         + bve_ref[:]) * maskf

    bd = bd_ref[:]
    parts = []
    for p in range(BG // PACK):
        s0, s1 = p * PM, (p + 1) * PM
        qp, kp, vp = q[s0:s1], k[s0:s1], v[s0:s1]
        energy = lax.dot_general(qp, kp, (((1,), (1,)), ((), ())),
                                 preferred_element_type=jnp.float32)
        energy = jnp.clip(energy, -50.0, 50.0)
        eexp = jnp.exp(energy) * bd
        we = jnp.dot(eexp, vp, preferred_element_type=jnp.float32)
        mo = we[:, :D] * (1.0 / we[:, D:D + 1])
        parts.append(0.5 * (mo * maskf[s0:s1]) + 0.5 * emb[s0:s1])
    overall = jnp.concatenate(parts, axis=0)

    ipart8 = jnp.dot(item_ref[:], w1b_ref[:],
                     preferred_element_type=jnp.float32)
    ipart = jnp.broadcast_to(ipart8[:, None, :],
                             (BG, MP, 16)).reshape(BG * MP, 16)
    h = jnp.maximum(
        jnp.dot(overall, w1a_ref[:], preferred_element_type=jnp.float32)
        + maskf * ipart + b1_ref[:], 0.0)
    a = jnp.dot(h, w2_ref[:], preferred_element_type=jnp.float32) + b2_ref[:]
    a = jnp.clip(a, -50.0, 50.0)
    aexp = jnp.exp(a) * maskf

    num = jnp.sum((aexp * overall).reshape(BG, MP, D), axis=1)
    den = jnp.sum(aexp.reshape(BG, MP, 1), axis=1)
    out_ref[:] = 0.5 * (num / den) + 0.5 * group_ref[:]


def _tc_attn(emb_flat, maskf, item_emb, group_emb,
             Wq, bq, Wk, bk, Wve, bve, W1a, W1b, b1, W2, b2, bd):
    full = lambda s: pl.BlockSpec(s, lambda i: (0, 0))
    return pl.pallas_call(
        _tc_attn_body,
        grid=(GRID,),
        in_specs=[
            pl.BlockSpec((BG * MP, D), lambda i: (i, 0)),
            pl.BlockSpec((BG * MP, 1), lambda i: (i, 0)),
            pl.BlockSpec((BG, D), lambda i: (i, 0)),
            pl.BlockSpec((BG, D), lambda i: (i, 0)),
            full((D, D)), full((1, D)),
            full((D, D)), full((1, D)),
            full((D, D + 1)), full((1, D + 1)),
            full((D, 16)), full((D, 16)), full((1, 16)),
            full((16, 1)), full((1, 1)),
            full((PM, PM)),
        ],
        out_specs=pl.BlockSpec((BG, D), lambda i: (i, 0)),
        out_shape=jax.ShapeDtypeStruct((B, D), jnp.float32),
    )(emb_flat, maskf, item_emb, group_emb,
      Wq, bq, Wk, bk, Wve, bve, W1a, W1b, b1, W2, b2, bd)


def kernel(gro_inputs, item_inputs, menb_ids, mask, u2e_w, v2e_w, g2e_w,
           Wq, bq, Wk, bk, Wv, bv, W1, b1, W2, b2):
    mask_pb = jnp.pad(mask, ((0, 0), (0, MP - M))) > 0.0
    mids_p = jnp.pad(menb_ids, ((0, 0), (0, MP - M)))
    spread = jax.lax.broadcasted_iota(jnp.int32, (B, MP), 0) * MP \
        + jax.lax.broadcasted_iota(jnp.int32, (B, MP), 1)
    mids_p = jnp.where(mask_pb, mids_p, spread % NU)
    mids2d = mids_p.reshape(ROWS // CH, CH)
    memb_flat, item_emb, group_emb = _sc_gather(
        mids2d, item_inputs, gro_inputs, u2e_w, v2e_w, g2e_w)

    mask_p = jnp.pad(mask, ((0, 0), (0, MP - M)))
    maskf = mask_p.reshape(B * MP, 1)
    Wve = jnp.pad(Wv, ((0, 0), (0, 1)))
    bve = jnp.pad(bv, ((0, 1)), constant_values=1.0).reshape(1, D + 1)
    bgid = jax.lax.broadcasted_iota(jnp.int32, (PM, PM), 0) // MP
    bd = (bgid == jax.lax.broadcasted_iota(jnp.int32, (PM, PM), 1) // MP
          ).astype(jnp.float32)
    return _tc_attn(memb_flat, maskf, item_emb, group_emb,
                    Wq, bq.reshape(1, D), Wk, bk.reshape(1, D),
                    Wve, bve,
                    W1[:D], W1[D:], b1.reshape(1, 16),
                    W2, b2.reshape(1, 1), bd)


# speedup vs baseline: 8.1415x; 1.0006x over previous
import functools

import jax
import jax.numpy as jnp
from jax import lax
from jax.experimental import pallas as pl
from jax.experimental.pallas import tpu as pltpu
from jax.experimental.pallas import tpu_sc as plsc

B = 4096
M = 50
MP = 64
D = 64
NU = 100000

NC = 2
NS = 16
NW = NC * NS

ROWS = B * MP
RPW = ROWS // NW
CH = 128
NCHUNK = RPW // CH
BPW = B // NW
NBUF = 4


def _sc_gather(mids2d, item_ids, gro_ids, u2e_w, v2e_w, g2e_w):
    mesh = plsc.VectorSubcoreMesh(core_axis_name="c", subcore_axis_name="s")

    @functools.partial(
        pl.kernel,
        mesh=mesh,
        out_type=[
            jax.ShapeDtypeStruct((ROWS, D), jnp.float32),
            jax.ShapeDtypeStruct((B, D), jnp.float32),
            jax.ShapeDtypeStruct((B, D), jnp.float32),
        ],
        scratch_types=[
            pltpu.VMEM((NCHUNK, CH), jnp.int32),
            [pltpu.VMEM((CH, D), jnp.float32) for _ in range(NBUF)],
            pltpu.VMEM((BPW,), jnp.int32),
            pltpu.VMEM((BPW, D), jnp.float32),
            [pltpu.SemaphoreType.DMA for _ in range(NBUF)],
            [pltpu.SemaphoreType.DMA for _ in range(NBUF)],
        ],
        compiler_params=pltpu.CompilerParams(use_tc_tiling_on_sc=False),
    )
    def k(mids_hbm, iids_hbm, gids_hbm, u2e_hbm, v2e_hbm, g2e_hbm,
          memb_out, item_out, group_out,
          idx_v, bufs, sid_v, rows_v, gsems, osems):
        wid = lax.axis_index("s") * NC + lax.axis_index("c")
        base = wid * RPW

        pltpu.sync_copy(mids_hbm.at[pl.ds(wid * NCHUNK, NCHUNK)], idx_v)

        def gather_start(c, b):
            pltpu.async_copy(u2e_hbm.at[idx_v.at[c]], bufs[b], gsems[b])

        def copyout_start(c, b):
            pltpu.async_copy(bufs[b],
                             memb_out.at[pl.ds(base + c * CH, CH)], osems[b])

        for b in range(NBUF):
            gather_start(b, b)

        def body(j):
            for b in range(NBUF):
                pltpu.make_async_copy(
                    u2e_hbm.at[idx_v.at[0]], bufs[b], gsems[b]).wait()
                copyout_start(j + b, b)
            for b in range(NBUF):
                nxt = j + b + NBUF

                @pl.when(nxt < NCHUNK)
                def _():
                    pltpu.make_async_copy(
                        bufs[b], memb_out.at[pl.ds(base, CH)], osems[b]).wait()
                    gather_start(nxt, b)

        pl.loop(0, NCHUNK, step=NBUF)(body)

        for b in range(NBUF):
            pltpu.make_async_copy(
                bufs[b], memb_out.at[pl.ds(base, CH)], osems[b]).wait()

        sbase = wid * BPW
        pltpu.sync_copy(iids_hbm.at[pl.ds(sbase, BPW)], sid_v)
        pltpu.async_copy(v2e_hbm.at[sid_v], rows_v, gsems[0]).wait()
        pltpu.sync_copy(rows_v, item_out.at[pl.ds(sbase, BPW)])

        pltpu.sync_copy(gids_hbm.at[pl.ds(sbase, BPW)], sid_v)
        pltpu.async_copy(g2e_hbm.at[sid_v], rows_v, gsems[0]).wait()
        pltpu.sync_copy(rows_v, group_out.at[pl.ds(sbase, BPW)])

    return k(mids2d, item_ids, gro_ids, u2e_w, v2e_w, g2e_w)


BG = 64
GRID = B // BG
PACK = 4
PM = PACK * MP


def _tc_attn_body(emb_ref, maskf_ref, item_ref, group_ref,
                  wq_ref, bq_ref, wk_ref, bk_ref, wve_ref, bve_ref,
                  w1a_ref, w1b_ref, b1_ref, w2_ref, b2_ref,
                  bd_ref, out_ref):
    maskf = maskf_ref[:]
    emb = emb_ref[:]
    q = jnp.dot(emb, wq_ref[:], preferred_element_type=jnp.float32) + bq_ref[:]
    k = jnp.dot(emb, wk_ref[:], preferred_element_type=jnp.float32) + bk_ref[:]
    v = (jnp.dot(emb, wve_ref[:], preferred_element_type=jnp.float32)
         + bve_ref[:]) * maskf

    bd = bd_ref[:]
    parts = []
    for p in range(BG // PACK):
        s0, s1 = p * PM, (p + 1) * PM
        qp, kp, vp = q[s0:s1], k[s0:s1], v[s0:s1]
        energy = lax.dot_general(qp, kp, (((1,), (1,)), ((), ())),
                                 preferred_element_type=jnp.float32)
        energy = jnp.clip(energy, -50.0, 50.0)
        eexp = jnp.exp(energy) * bd
        we = jnp.dot(eexp, vp, preferred_element_type=jnp.float32)
        mo = we[:, :D] * (1.0 / we[:, D:D + 1])
        parts.append(mo + emb[s0:s1])
    overall2 = jnp.concatenate(parts, axis=0)

    ipart8 = jnp.dot(item_ref[:], w1b_ref[:],
                     preferred_element_type=jnp.float32)
    ipart = jnp.broadcast_to(ipart8[:, None, :],
                             (BG, MP, 16)).reshape(BG * MP, 16)
    h = jnp.maximum(
        jnp.dot(overall2, w1a_ref[:], preferred_element_type=jnp.float32)
        + maskf * ipart + b1_ref[:], 0.0)
    a = jnp.dot(h, w2_ref[:], preferred_element_type=jnp.float32) + b2_ref[:]
    a = jnp.clip(a, -50.0, 50.0)
    aexp = jnp.exp(a) * maskf

    num = jnp.sum((aexp * overall2).reshape(BG, MP, D), axis=1)
    den = jnp.sum(aexp.reshape(BG, MP, 1), axis=1)
    out_ref[:] = 0.25 * (num / den) + 0.5 * group_ref[:]


def _tc_attn(emb_flat, maskf, item_emb, group_emb,
             Wq, bq, Wk, bk, Wve, bve, W1a, W1b, b1, W2, b2, bd):
    full = lambda s: pl.BlockSpec(s, lambda i: (0, 0))
    return pl.pallas_call(
        _tc_attn_body,
        grid=(GRID,),
        in_specs=[
            pl.BlockSpec((BG * MP, D), lambda i: (i, 0)),
            pl.BlockSpec((BG * MP, 1), lambda i: (i, 0)),
            pl.BlockSpec((BG, D), lambda i: (i, 0)),
            pl.BlockSpec((BG, D), lambda i: (i, 0)),
            full((D, D)), full((1, D)),
            full((D, D)), full((1, D)),
            full((D, D + 1)), full((1, D + 1)),
            full((D, 16)), full((D, 16)), full((1, 16)),
            full((16, 1)), full((1, 1)),
            full((PM, PM)),
        ],
        out_specs=pl.BlockSpec((BG, D), lambda i: (i, 0)),
        out_shape=jax.ShapeDtypeStruct((B, D), jnp.float32),
    )(emb_flat, maskf, item_emb, group_emb,
      Wq, bq, Wk, bk, Wve, bve, W1a, W1b, b1, W2, b2, bd)


def kernel(gro_inputs, item_inputs, menb_ids, mask, u2e_w, v2e_w, g2e_w,
           Wq, bq, Wk, bk, Wv, bv, W1, b1, W2, b2):
    mask_pb = jnp.pad(mask, ((0, 0), (0, MP - M))) > 0.0
    mids_p = jnp.pad(menb_ids, ((0, 0), (0, MP - M)))
    spread = jax.lax.broadcasted_iota(jnp.int32, (B, MP), 0) * MP \
        + jax.lax.broadcasted_iota(jnp.int32, (B, MP), 1)
    mids_p = jnp.where(mask_pb, mids_p, spread % NU)
    mids2d = mids_p.reshape(ROWS // CH, CH)
    memb_flat, item_emb, group_emb = _sc_gather(
        mids2d, item_inputs, gro_inputs, u2e_w, v2e_w, g2e_w)

    mask_p = jnp.pad(mask, ((0, 0), (0, MP - M)))
    maskf = mask_p.reshape(B * MP, 1)
    Wve = jnp.pad(Wv, ((0, 0), (0, 1)))
    bve = jnp.pad(bv, ((0, 1)), constant_values=1.0).reshape(1, D + 1)
    bgid = jax.lax.broadcasted_iota(jnp.int32, (PM, PM), 0) // MP
    bd = (bgid == jax.lax.broadcasted_iota(jnp.int32, (PM, PM), 1) // MP
          ).astype(jnp.float32)
    return _tc_attn(memb_flat, maskf, item_emb, group_emb,
                    Wq, bq.reshape(1, D), Wk, bk.reshape(1, D),
                    Wve, bve,
                    0.5 * W1[:D], W1[D:], b1.reshape(1, 16),
                    W2, b2.reshape(1, 1), bd)


# speedup vs baseline: 10.7259x; 1.3174x over previous
import functools

import jax
import jax.numpy as jnp
from jax import lax
from jax.experimental import pallas as pl
from jax.experimental.pallas import tpu as pltpu
from jax.experimental.pallas import tpu_sc as plsc

B = 4096
M = 50
MP = 64
D = 64
NU = 100000

NC = 2
NS = 16
NW = NC * NS

ROWS = B * MP
RPW = ROWS // NW
CH = 128
NCHUNK = RPW // CH
BPW = B // NW
NBUF = 4


def _sc_gather(mids2d, item_ids, gro_ids, u2e_w, v2e_w, g2e_w):
    mesh = plsc.VectorSubcoreMesh(core_axis_name="c", subcore_axis_name="s")

    @functools.partial(
        pl.kernel,
        mesh=mesh,
        out_type=[
            jax.ShapeDtypeStruct((ROWS, D), jnp.float32),
            jax.ShapeDtypeStruct((B, D), jnp.float32),
            jax.ShapeDtypeStruct((B, D), jnp.float32),
        ],
        scratch_types=[
            pltpu.VMEM((NCHUNK, CH), jnp.int32),
            [pltpu.VMEM((CH, D), jnp.float32) for _ in range(NBUF)],
            pltpu.VMEM((BPW,), jnp.int32),
            pltpu.VMEM((BPW, D), jnp.float32),
            [pltpu.SemaphoreType.DMA for _ in range(NBUF)],
            [pltpu.SemaphoreType.DMA for _ in range(NBUF)],
        ],
        compiler_params=pltpu.CompilerParams(use_tc_tiling_on_sc=False),
    )
    def k(mids_hbm, iids_hbm, gids_hbm, u2e_hbm, v2e_hbm, g2e_hbm,
          memb_out, item_out, group_out,
          idx_v, bufs, sid_v, rows_v, gsems, osems):
        wid = lax.axis_index("s") * NC + lax.axis_index("c")
        base = wid * RPW

        pltpu.sync_copy(mids_hbm.at[pl.ds(wid * NCHUNK, NCHUNK)], idx_v)

        def gather_start(c, b):
            pltpu.async_copy(u2e_hbm.at[idx_v.at[c]], bufs[b], gsems[b])

        def copyout_start(c, b):
            pltpu.async_copy(bufs[b],
                             memb_out.at[pl.ds(base + c * CH, CH)], osems[b])

        for b in range(NBUF):
            gather_start(b, b)

        def body(j):
            for b in range(NBUF):
                pltpu.make_async_copy(
                    u2e_hbm.at[idx_v.at[0]], bufs[b], gsems[b]).wait()
                copyout_start(j + b, b)
            for b in range(NBUF):
                nxt = j + b + NBUF

                @pl.when(nxt < NCHUNK)
                def _():
                    pltpu.make_async_copy(
                        bufs[b], memb_out.at[pl.ds(base, CH)], osems[b]).wait()
                    gather_start(nxt, b)

        pl.loop(0, NCHUNK, step=NBUF)(body)

        for b in range(NBUF):
            pltpu.make_async_copy(
                bufs[b], memb_out.at[pl.ds(base, CH)], osems[b]).wait()

        sbase = wid * BPW
        pltpu.sync_copy(iids_hbm.at[pl.ds(sbase, BPW)], sid_v)
        pltpu.async_copy(v2e_hbm.at[sid_v], rows_v, gsems[0]).wait()
        pltpu.sync_copy(rows_v, item_out.at[pl.ds(sbase, BPW)])

        pltpu.sync_copy(gids_hbm.at[pl.ds(sbase, BPW)], sid_v)
        pltpu.async_copy(g2e_hbm.at[sid_v], rows_v, gsems[0]).wait()
        pltpu.sync_copy(rows_v, group_out.at[pl.ds(sbase, BPW)])

    return k(mids2d, item_ids, gro_ids, u2e_w, v2e_w, g2e_w)


BG = 64
GRID = B // BG
PACK = 4
PM = PACK * MP


def _tc_attn_body(emb_ref, mask_ref, item_ref, group_ref,
                  wq_ref, bq_ref, wk_ref, bk_ref, wve_ref, bve_ref,
                  w1a_ref, w1b_ref, b1_ref, w2_ref, b2_ref,
                  bd_ref, out_ref):
    maskf = jnp.transpose(mask_ref[:])
    e2 = emb_ref[:]
    lo = e2[:, :D].reshape(BG, MP // 2, D)
    hi = e2[:, D:].reshape(BG, MP // 2, D)
    emb = jnp.concatenate([lo, hi], axis=1).reshape(BG * MP, D)
    q = jnp.dot(emb, wq_ref[:], preferred_element_type=jnp.float32) + bq_ref[:]
    k = jnp.dot(emb, wk_ref[:], preferred_element_type=jnp.float32) + bk_ref[:]
    v = (jnp.dot(emb, wve_ref[:], preferred_element_type=jnp.float32)
         + bve_ref[:]) * maskf

    bd = bd_ref[:]
    parts = []
    for p in range(BG // PACK):
        s0, s1 = p * PM, (p + 1) * PM
        qp, kp, vp = q[s0:s1], k[s0:s1], v[s0:s1]
        energy = lax.dot_general(qp, kp, (((1,), (1,)), ((), ())),
                                 preferred_element_type=jnp.float32)
        energy = jnp.clip(energy, -50.0, 50.0)
        eexp = jnp.exp(energy) * bd
        we = jnp.dot(eexp, vp, preferred_element_type=jnp.float32)
        mo = we[:, :D] * (1.0 / we[:, D:D + 1])
        parts.append(mo + emb[s0:s1])
    overall2 = jnp.concatenate(parts, axis=0)

    ipart8 = jnp.dot(item_ref[:], w1b_ref[:],
                     preferred_element_type=jnp.float32)
    ipart = jnp.broadcast_to(ipart8[:, None, :],
                             (BG, MP, 16)).reshape(BG * MP, 16)
    h = jnp.maximum(
        jnp.dot(overall2, w1a_ref[:], preferred_element_type=jnp.float32)
        + maskf * ipart + b1_ref[:], 0.0)
    a = jnp.dot(h, w2_ref[:], preferred_element_type=jnp.float32) + b2_ref[:]
    a = jnp.clip(a, -50.0, 50.0)
    aexp = jnp.exp(a) * maskf

    num = jnp.sum((aexp * overall2).reshape(BG, MP, D), axis=1)
    den = jnp.sum(aexp.reshape(BG, MP, 1), axis=1)
    out_ref[:] = 0.25 * (num / den) + 0.5 * group_ref[:]


def _tc_attn(emb_flat, maskf, item_emb, group_emb,
             Wq, bq, Wk, bk, Wve, bve, W1a, W1b, b1, W2, b2, bd):
    full = lambda s: pl.BlockSpec(s, lambda i: (0, 0))
    return pl.pallas_call(
        _tc_attn_body,
        grid=(GRID,),
        in_specs=[
            pl.BlockSpec((BG * MP // 2, 2 * D), lambda i: (i, 0)),
            pl.BlockSpec((1, BG * MP), lambda i: (0, i)),
            pl.BlockSpec((BG, D), lambda i: (i, 0)),
            pl.BlockSpec((BG, D), lambda i: (i, 0)),
            full((D, D)), full((1, D)),
            full((D, D)), full((1, D)),
            full((D, D + 1)), full((1, D + 1)),
            full((D, 16)), full((D, 16)), full((1, 16)),
            full((16, 1)), full((1, 1)),
            full((PM, PM)),
        ],
        out_specs=pl.BlockSpec((BG, D), lambda i: (i, 0)),
        out_shape=jax.ShapeDtypeStruct((B, D), jnp.float32),
    )(emb_flat, maskf, item_emb, group_emb,
      Wq, bq, Wk, bk, Wve, bve, W1a, W1b, b1, W2, b2, bd)


def kernel(gro_inputs, item_inputs, menb_ids, mask, u2e_w, v2e_w, g2e_w,
           Wq, bq, Wk, bk, Wv, bv, W1, b1, W2, b2):
    mask_pb = jnp.pad(mask, ((0, 0), (0, MP - M))) > 0.0
    mids_p = jnp.pad(menb_ids, ((0, 0), (0, MP - M)))
    spread = jax.lax.broadcasted_iota(jnp.int32, (B, MP), 0) * MP \
        + jax.lax.broadcasted_iota(jnp.int32, (B, MP), 1)
    mids_p = jnp.where(mask_pb, mids_p, spread % NU)
    mids2d = mids_p.reshape(ROWS // CH, CH)
    memb_flat, item_emb, group_emb = _sc_gather(
        mids2d, item_inputs, gro_inputs, u2e_w, v2e_w, g2e_w)

    mask_p = jnp.pad(mask, ((0, 0), (0, MP - M)))
    memb2 = memb_flat.reshape(ROWS // 2, 2 * D)
    Wve = jnp.pad(Wv, ((0, 0), (0, 1)))
    bve = jnp.pad(bv, ((0, 1)), constant_values=1.0).reshape(1, D + 1)
    bgid = jax.lax.broadcasted_iota(jnp.int32, (PM, PM), 0) // MP
    bd = (bgid == jax.lax.broadcasted_iota(jnp.int32, (PM, PM), 1) // MP
          ).astype(jnp.float32)
    perm = jnp.concatenate([jnp.arange(0, MP, 2), jnp.arange(1, MP, 2)])
    mask_perm = mask_p[:, perm]
    return _tc_attn(memb2, mask_perm.reshape(1, ROWS), item_emb,
                    group_emb,
                    Wq, bq.reshape(1, D), Wk, bk.reshape(1, D),
                    Wve, bve,
                    0.5 * W1[:D], W1[D:], b1.reshape(1, 16),
                    W2, b2.reshape(1, 1), bd)
